# baseline (speedup 1.0000x reference)
import functools

import jax
import jax.numpy as jnp
from jax import lax
from jax.experimental import pallas as pl
from jax.experimental.pallas import tpu as pltpu
from jax.experimental.pallas import tpu_sc as plsc

B = 16384
E = 128
NC = 2
NS = 16
NW = NC * NS
BPW = B // NW
CHUNK = 128
NCHUNK = BPW // CHUNK
GROUPS = CHUNK // 16


def _mf_body(u_hbm, v_hbm, ue_hbm, ie_hbm, ub_hbm, ib_hbm, out_hbm,
             idx_u, idx_v, u_rows, v_rows, bu, bv, res,
             sem_r0, sem_r1, sem_b):
    wid = lax.axis_index("s") * NC + lax.axis_index("c")
    base = wid * BPW

    def fire(c, p, sem):
        sl = pl.ds(c * CHUNK, CHUNK)
        pltpu.async_copy(ue_hbm.at[idx_u.at[sl]], u_rows.at[p], sem)
        pltpu.async_copy(ie_hbm.at[idx_v.at[sl]], v_rows.at[p], sem)

    def drain(p, sem):
        pltpu.make_async_copy(ue_hbm.at[pl.ds(0, CHUNK)], u_rows.at[p], sem).wait()
        pltpu.make_async_copy(ie_hbm.at[pl.ds(0, CHUNK)], v_rows.at[p], sem).wait()

    pltpu.sync_copy(u_hbm.at[pl.ds(base, CHUNK)], idx_u.at[pl.ds(0, CHUNK)])
    pltpu.sync_copy(v_hbm.at[pl.ds(base, CHUNK)], idx_v.at[pl.ds(0, CHUNK)])
    fire(0, 0, sem_r0)
    pltpu.sync_copy(u_hbm.at[pl.ds(base + CHUNK, BPW - CHUNK)],
                    idx_u.at[pl.ds(CHUNK, BPW - CHUNK)])
    pltpu.sync_copy(v_hbm.at[pl.ds(base + CHUNK, BPW - CHUNK)],
                    idx_v.at[pl.ds(CHUNK, BPW - CHUNK)])
    fire(1, 1, sem_r1)

    bias_cps = []
    for c in range(BPW // 128):
        sl = pl.ds(c * 128, 128)
        bias_cps.append(
            pltpu.async_copy(ub_hbm.at[idx_u.at[sl]], bu.at[sl], sem_b))
        bias_cps.append(
            pltpu.async_copy(ib_hbm.at[idx_v.at[sl]], bv.at[sl], sem_b))

    lane = lax.iota(jnp.int32, 16)

    def compute_chunk(c, p):
        def group_body(g, _):
            base_row = g * 16

            def row_body(r, tot):
                row = base_row + r
                acc = (u_rows[p, row, pl.ds(0, 16)]
                       * v_rows[p, row, pl.ds(0, 16)])
                for j in range(1, E // 16):
                    acc = acc + (u_rows[p, row, pl.ds(j * 16, 16)]
                                 * v_rows[p, row, pl.ds(j * 16, 16)])
                return jnp.where(lane == r, jnp.sum(acc), tot)

            tot = lax.fori_loop(0, 16, row_body,
                                jnp.zeros((16,), jnp.float32), unroll=4)
            res[pl.ds(c * CHUNK + base_row, 16)] = tot
            return 0

        lax.fori_loop(0, GROUPS, group_body, 0)

    def super_body(i, _):
        c0 = 2 * i
        drain(0, sem_r0)
        compute_chunk(c0, 0)

        @pl.when(c0 + 2 < NCHUNK)
        def _():
            fire(c0 + 2, 0, sem_r0)

        drain(1, sem_r1)
        compute_chunk(c0 + 1, 1)

        @pl.when(c0 + 3 < NCHUNK)
        def _():
            fire(c0 + 3, 1, sem_r1)

        return 0

    lax.fori_loop(0, NCHUNK // 2, super_body, 0)

    for d in bias_cps:
        d.wait()

    def fin_body(i, _):
        s = pl.ds(i * 16, 16)
        x = res[s] + bu[s] + bv[s]
        res[s] = 1.0 / (1.0 + jnp.exp(-x))
        return 0

    lax.fori_loop(0, BPW // 16, fin_body, 0)
    pltpu.sync_copy(res, out_hbm.at[pl.ds(base, BPW)])


@functools.partial(
    pl.kernel,
    out_type=jax.ShapeDtypeStruct((B,), jnp.float32),
    mesh=plsc.VectorSubcoreMesh(core_axis_name="c", subcore_axis_name="s"),
    compiler_params=pltpu.CompilerParams(needs_layout_passes=False,
                                         disable_bounds_checks=True,
                                         skip_device_barrier=True),
    scratch_types=[
        pltpu.VMEM((BPW,), jnp.int32),
        pltpu.VMEM((BPW,), jnp.int32),
        pltpu.VMEM((2, CHUNK, E), jnp.float32),
        pltpu.VMEM((2, CHUNK, E), jnp.float32),
        pltpu.VMEM((BPW,), jnp.float32),
        pltpu.VMEM((BPW,), jnp.float32),
        pltpu.VMEM((BPW,), jnp.float32),
        pltpu.SemaphoreType.DMA,
        pltpu.SemaphoreType.DMA,
        pltpu.SemaphoreType.DMA,
    ],
)
def _mf(*refs):
    _mf_body(*refs)


def kernel(u, v, user_emb_w, item_emb_w, user_bias_w, item_bias_w):
    ub = user_bias_w.reshape((-1,))
    ib = item_bias_w.reshape((-1,))
    return _mf(u.astype(jnp.int32), v.astype(jnp.int32),
               user_emb_w, item_emb_w, ub, ib)


# speedup vs baseline: 1.0042x; 1.0042x over previous
import functools

import jax
import jax.numpy as jnp
from jax import lax
from jax.experimental import pallas as pl
from jax.experimental.pallas import tpu as pltpu
from jax.experimental.pallas import tpu_sc as plsc

B = 16384
E = 128
NC = 2
NS = 16
NW = NC * NS
BPW = B // NW
CHUNK = 128
NCHUNK = BPW // CHUNK
GROUPS = CHUNK // 16


def _mf_body(u_hbm, v_hbm, ue_hbm, ie_hbm, ub_hbm, ib_hbm, out_hbm,
             idx_u, idx_v, u_rows, v_rows, bu, bv, res,
             sem_r0, sem_r1, sem_b):
    wid = lax.axis_index("s") * NC + lax.axis_index("c")
    base = wid * BPW

    def fire(c, p, sem):
        sl = pl.ds(c * CHUNK, CHUNK)
        pltpu.async_copy(ue_hbm.at[idx_u.at[sl]], u_rows.at[p], sem)
        pltpu.async_copy(ie_hbm.at[idx_v.at[sl]], v_rows.at[p], sem)

    def drain(p, sem):
        pltpu.make_async_copy(ue_hbm.at[pl.ds(0, CHUNK)], u_rows.at[p], sem).wait()
        pltpu.make_async_copy(ie_hbm.at[pl.ds(0, CHUNK)], v_rows.at[p], sem).wait()

    pltpu.sync_copy(u_hbm.at[pl.ds(base, CHUNK)], idx_u.at[pl.ds(0, CHUNK)])
    pltpu.sync_copy(v_hbm.at[pl.ds(base, CHUNK)], idx_v.at[pl.ds(0, CHUNK)])
    fire(0, 0, sem_r0)
    pltpu.sync_copy(u_hbm.at[pl.ds(base + CHUNK, BPW - CHUNK)],
                    idx_u.at[pl.ds(CHUNK, BPW - CHUNK)])
    pltpu.sync_copy(v_hbm.at[pl.ds(base + CHUNK, BPW - CHUNK)],
                    idx_v.at[pl.ds(CHUNK, BPW - CHUNK)])
    fire(1, 1, sem_r1)

    bias_cps = []
    for c in range(BPW // 128):
        sl = pl.ds(c * 128, 128)
        bias_cps.append(
            pltpu.async_copy(ub_hbm.at[idx_u.at[sl]], bu.at[sl], sem_b))
        bias_cps.append(
            pltpu.async_copy(ib_hbm.at[idx_v.at[sl]], bv.at[sl], sem_b))

    lane = lax.iota(jnp.int32, 16)

    def compute_chunk(c, p):
        def group_body(g, _):
            base_row = g * 16

            def row_body(r, tot):
                row = base_row + r
                acc = (u_rows[p, row, pl.ds(0, 16)]
                       * v_rows[p, row, pl.ds(0, 16)])
                for j in range(1, E // 16):
                    acc = acc + (u_rows[p, row, pl.ds(j * 16, 16)]
                                 * v_rows[p, row, pl.ds(j * 16, 16)])
                return jnp.where(lane == r, jnp.sum(acc), tot)

            tot = lax.fori_loop(0, 16, row_body,
                                jnp.zeros((16,), jnp.float32), unroll=4)
            res[pl.ds(c * CHUNK + base_row, 16)] = tot
            return 0

        lax.fori_loop(0, GROUPS, group_body, 0)

    def super_body(i, _):
        c0 = 2 * i
        drain(0, sem_r0)
        compute_chunk(c0, 0)

        @pl.when(c0 + 2 < NCHUNK)
        def _():
            fire(c0 + 2, 0, sem_r0)

        drain(1, sem_r1)
        compute_chunk(c0 + 1, 1)

        @pl.when(c0 + 3 < NCHUNK)
        def _():
            fire(c0 + 3, 1, sem_r1)

        return 0

    lax.fori_loop(0, NCHUNK // 2, super_body, 0)

    for d in bias_cps:
        d.wait()

    def fin_body(i, _):
        s = pl.ds(i * 16, 16)
        x = res[s] + bu[s] + bv[s]
        res[s] = 1.0 / (1.0 + jnp.exp(-x))
        return 0

    lax.fori_loop(0, BPW // 16, fin_body, 0)
    pltpu.sync_copy(res, out_hbm.at[pl.ds(base, BPW)])


@functools.partial(
    pl.kernel,
    out_type=jax.ShapeDtypeStruct((B,), jnp.float32),
    mesh=plsc.VectorSubcoreMesh(core_axis_name="c", subcore_axis_name="s"),
    compiler_params=pltpu.CompilerParams(needs_layout_passes=False),
    scratch_types=[
        pltpu.VMEM((BPW,), jnp.int32),
        pltpu.VMEM((BPW,), jnp.int32),
        pltpu.VMEM((2, CHUNK, E), jnp.float32),
        pltpu.VMEM((2, CHUNK, E), jnp.float32),
        pltpu.VMEM((BPW,), jnp.float32),
        pltpu.VMEM((BPW,), jnp.float32),
        pltpu.VMEM((BPW,), jnp.float32),
        pltpu.SemaphoreType.DMA,
        pltpu.SemaphoreType.DMA,
        pltpu.SemaphoreType.DMA,
    ],
)
def _mf(*refs):
    _mf_body(*refs)


def kernel(u, v, user_emb_w, item_emb_w, user_bias_w, item_bias_w):
    ub = user_bias_w.reshape((-1,))
    ib = item_bias_w.reshape((-1,))
    return _mf(u.astype(jnp.int32), v.astype(jnp.int32),
               user_emb_w, item_emb_w, ub, ib)
